# Initial kernel scaffold; baseline (speedup 1.0000x reference)
import jax
import jax.numpy as jnp
from jax.experimental import pallas as pl


def kernel(fe, lengths):
    raise NotImplementedError("write your pallas kernel here")


# speedup vs baseline: 8.5260x; 8.5260x over previous
import jax
import jax.numpy as jnp
from jax import lax
from jax.experimental import pallas as pl
from jax.experimental.pallas import tpu as pltpu

_TARGET = 1536
_B, _C, _E = 8, 256, 2048
_JC = 256
_TC = 512


def _pool_body(len_ref, fe_ref, out_ref):
    b = pl.program_id(0)
    length = len_ref[b]
    k = length - _TARGET
    fe = fe_ref[...]
    sc = jnp.sum(fe * fe, axis=0, keepdims=True)
    eidx_row = lax.broadcasted_iota(jnp.int32, (1, _E), 1)
    sc = jnp.where(eidx_row < length, sc, jnp.float32(jnp.inf))
    scT = lax.transpose(jnp.broadcast_to(sc, (8, _E)), (1, 0))
    sc_col = scT[:, 0:1]
    jidx_col = lax.broadcasted_iota(jnp.int32, (_E, 1), 0)
    rank = jnp.zeros((1, _E), jnp.int32)
    for jc in range(_E // _JC):
        sj = sc_col[jc * _JC:(jc + 1) * _JC, :]
        ji = jidx_col[jc * _JC:(jc + 1) * _JC, :]
        less = (sj < sc) | ((sj == sc) & (ji < eidx_row))
        rank = rank + jnp.sum(less.astype(jnp.int32), axis=0, keepdims=True)
    dst = jnp.where(rank < k, rank, rank - k)
    dst = jnp.where(eidx_row < length, dst, -1)
    for tc in range(_TARGET // _TC):
        tval = lax.broadcasted_iota(jnp.int32, (_TC, 1), 0) + tc * _TC
        pt = (tval == dst).astype(jnp.float32)
        p = lax.transpose(pt, (1, 0))
        out_ref[:, tc * _TC:(tc + 1) * _TC] = jnp.dot(
            fe, p, preferred_element_type=jnp.float32)


def kernel(fe, lengths):
    return pl.pallas_call(
        _pool_body,
        grid=(_B,),
        in_specs=[
            pl.BlockSpec(memory_space=pltpu.SMEM),
            pl.BlockSpec((None, _C, _E), lambda b: (b, 0, 0)),
        ],
        out_specs=pl.BlockSpec((None, _C, _TARGET), lambda b: (b, 0, 0)),
        out_shape=jax.ShapeDtypeStruct((_B, _C, _TARGET), jnp.float32),
    )(lengths, fe)
